# Initial kernel scaffold; baseline (speedup 1.0000x reference)
import jax
import jax.numpy as jnp
from jax.experimental import pallas as pl


def kernel(x, edge_index, edge_attr, batch, bn0_g, bn0_b, bn1_g, bn1_b, bn2_g, bn2_b, bn3_g, bn3_b, e1_W, e1_b, m1_W1, m1_b1, m1_g, m1_bb, m1_W2, m1_b2, e2_W, e2_b, m2_W1, m2_b1, m2_g, m2_bb, m2_W2, m2_b2, e3_W, e3_b, m3_W1, m3_b1, m3_g, m3_bb, m3_W2, m3_b2):
    raise NotImplementedError("write your pallas kernel here")


# speedup vs baseline: 2.4399x; 2.4399x over previous
import functools

import jax
import jax.numpy as jnp
from jax import lax
from jax.experimental import pallas as pl
from jax.experimental.pallas import tpu as pltpu
from jax.experimental.pallas import tpu_sc as plsc

N = 10000
E = 320000
D = 128
ED = 16
G = 64

NC = 2
NS = 16
NW = NC * NS
EPW = E // NW
C = 80
NCHUNK = EPW // C
NPAD = 10240
RPS = NPAD // NS
RZ = 128


def _bn_body(x_ref, g_ref, b_ref, o_ref):
    x = x_ref[...]
    m = jnp.mean(x, axis=0, keepdims=True)
    v = jnp.mean((x - m) ** 2, axis=0, keepdims=True)
    o_ref[...] = (x - m) * jax.lax.rsqrt(v + 1e-5) * g_ref[...] + b_ref[...]


def _bn0(x, g, b):
    return pl.pallas_call(
        _bn_body,
        out_shape=jax.ShapeDtypeStruct((N, D), jnp.float32),
    )(x, g.reshape(1, D), b.reshape(1, D))


_EBLK = 4000


def _eproj_body(ea_ref, w1_ref, b1_ref, w2_ref, b2_ref, w3_ref, b3_ref,
                o1_ref, o2_ref, o3_ref):
    ea = ea_ref[...]
    dn = (((1,), (1,)), ((), ()))
    o1_ref[...] = lax.dot_general(ea, w1_ref[...], dn,
                                  preferred_element_type=jnp.float32) + b1_ref[...]
    o2_ref[...] = lax.dot_general(ea, w2_ref[...], dn,
                                  preferred_element_type=jnp.float32) + b2_ref[...]
    o3_ref[...] = lax.dot_general(ea, w3_ref[...], dn,
                                  preferred_element_type=jnp.float32) + b3_ref[...]


def _eproj(ea, w1, b1, w2, b2, w3, b3):
    nblk = E // _EBLK
    wspec = pl.BlockSpec((D, ED), lambda i: (0, 0))
    bspec = pl.BlockSpec((1, D), lambda i: (0, 0))
    ospec = pl.BlockSpec((_EBLK, D), lambda i: (i, 0))
    return pl.pallas_call(
        _eproj_body,
        grid=(nblk,),
        in_specs=[pl.BlockSpec((_EBLK, ED), lambda i: (i, 0)),
                  wspec, bspec, wspec, bspec, wspec, bspec],
        out_specs=[ospec, ospec, ospec],
        out_shape=[jax.ShapeDtypeStruct((E, D), jnp.float32)] * 3,
    )(ea, w1, b1.reshape(1, D), w2, b2.reshape(1, D), w3, b3.reshape(1, D))


def _mlp_body(final_relu, h_ref, p0_ref, p1_ref, w1_ref, b1_ref, g1_ref,
              bb1_ref, w2_ref, b2_ref, og_ref, ob_ref, o_ref):
    y = h_ref[...] + p0_ref[...] + p1_ref[...]
    dn = (((1,), (1,)), ((), ()))
    t = lax.dot_general(y, w1_ref[...], dn,
                        preferred_element_type=jnp.float32) + b1_ref[...]
    m = jnp.mean(t, axis=0, keepdims=True)
    v = jnp.mean((t - m) ** 2, axis=0, keepdims=True)
    t = (t - m) * lax.rsqrt(v + 1e-5) * g1_ref[...] + bb1_ref[...]
    t = jnp.maximum(t, 0.0)
    u = lax.dot_general(t, w2_ref[...], dn,
                        preferred_element_type=jnp.float32) + b2_ref[...]
    m2 = jnp.mean(u, axis=0, keepdims=True)
    v2 = jnp.mean((u - m2) ** 2, axis=0, keepdims=True)
    u = (u - m2) * lax.rsqrt(v2 + 1e-5) * og_ref[...] + ob_ref[...]
    if final_relu:
        u = jnp.maximum(u, 0.0)
    o_ref[...] = u


def _mlp(h, p0, p1, w1, b1, g1, bb1, w2, b2, og, ob, final_relu):
    return pl.pallas_call(
        functools.partial(_mlp_body, final_relu),
        out_shape=jax.ShapeDtypeStruct((N, D), jnp.float32),
    )(h, p0, p1, w1, b1.reshape(1, D), g1.reshape(1, D), bb1.reshape(1, D),
      w2, b2.reshape(1, D), og.reshape(1, D), ob.reshape(1, D))


def _pool_body(h_ref, bat_ref, o_ref):
    h = h_ref[...]
    bat = bat_ref[...]
    iota = lax.broadcasted_iota(jnp.int32, (N, G), 1)
    oh = (bat == iota).astype(jnp.float32)
    counts = jnp.sum(oh, axis=0, keepdims=True)
    ssum = lax.dot_general(oh, h, (((0,), (0,)), ((), ())),
                           preferred_element_type=jnp.float32)
    mean = ssum * (1.0 / jnp.maximum(counts, 1.0)).reshape(G, 1)
    rows = []
    for g in range(G):
        mg = jnp.max(jnp.where(bat == g, h, -jnp.inf), axis=0, keepdims=True)
        rows.append(mg)
    mx = jnp.concatenate(rows, axis=0)
    o_ref[:, :D] = mean
    o_ref[:, D:] = mx


def _pool(h, batch):
    return pl.pallas_call(
        _pool_body,
        out_shape=jax.ShapeDtypeStruct((G, 2 * D), jnp.float32),
    )(h, batch.reshape(N, 1))


def _sc_agg_body(x_hbm, src_hbm, dst_hbm, ep_hbm, out_hbm,
                 src_v, dst_v, ep_v, xr_v, z_v, acc_sh, sem):
    c = lax.axis_index("c")
    s = lax.axis_index("s")
    w = c * NS + s
    base = w * EPW

    zero16 = jnp.zeros((16,), jnp.float32)

    def zrow(i, carry):
        for j in range(D // 16):
            z_v[i, pl.ds(j * 16, 16)] = zero16
        return carry

    lax.fori_loop(0, RZ, zrow, 0)
    for r in range(RPS // RZ):
        pltpu.sync_copy(z_v, acc_sh.at[pl.ds(s * RPS + r * RZ, RZ)])
    plsc.subcore_barrier()

    def chunk(gi, carry):
        off = base + gi * C
        pltpu.sync_copy(src_hbm.at[pl.ds(off, C)], src_v)
        pltpu.sync_copy(dst_hbm.at[pl.ds(off, C)], dst_v)
        pltpu.sync_copy(ep_hbm.at[pl.ds(off, C)], ep_v)
        pltpu.async_copy(x_hbm.at[src_v], xr_v, sem).wait()

        def edge(i, carry2):
            for j in range(D // 16):
                sl = pl.ds(j * 16, 16)
                xr_v[i, sl] = jnp.maximum(xr_v[i, sl] + ep_v[i, sl], 0.0)
            return carry2

        lax.fori_loop(0, C, edge, 0)
        pltpu.sync_copy(xr_v, acc_sh.at[dst_v], add=True)
        return carry

    lax.fori_loop(0, NCHUNK, chunk, 0)
    plsc.subcore_barrier()

    for r in range(RPS // RZ):
        row0 = s * RPS + r * RZ
        pltpu.sync_copy(acc_sh.at[pl.ds(row0, RZ)], z_v)
        pltpu.sync_copy(z_v, out_hbm.at[c, pl.ds(row0, RZ)])


def _sc_agg(x, src, dst, ep):
    mesh = plsc.VectorSubcoreMesh(core_axis_name="c", subcore_axis_name="s")
    f = pl.kernel(
        _sc_agg_body,
        out_type=jax.ShapeDtypeStruct((NC, NPAD, D), jnp.float32),
        mesh=mesh,
        scratch_types=[
            pltpu.VMEM((C,), jnp.int32),
            pltpu.VMEM((C,), jnp.int32),
            pltpu.VMEM((C, D), jnp.float32),
            pltpu.VMEM((C, D), jnp.float32),
            pltpu.VMEM((RZ, D), jnp.float32),
            pltpu.VMEM_SHARED((NPAD, D), jnp.float32),
            pltpu.SemaphoreType.DMA,
        ],
    )
    return f(x, src, dst, ep)


def kernel(x, edge_index, edge_attr, batch,
           bn0_g, bn0_b, bn1_g, bn1_b, bn2_g, bn2_b, bn3_g, bn3_b,
           e1_W, e1_b, m1_W1, m1_b1, m1_g, m1_bb, m1_W2, m1_b2,
           e2_W, e2_b, m2_W1, m2_b1, m2_g, m2_bb, m2_W2, m2_b2,
           e3_W, e3_b, m3_W1, m3_b1, m3_g, m3_bb, m3_W2, m3_b2):
    src = edge_index[0]
    dst = edge_index[1]

    h = _bn0(x, bn0_g, bn0_b)
    ep1, ep2, ep3 = _eproj(edge_attr, e1_W, e1_b, e2_W, e2_b, e3_W, e3_b)

    p = _sc_agg(h, src, dst, ep1)[:, :N]
    h = _mlp(h, p[0], p[1], m1_W1, m1_b1, m1_g, m1_bb, m1_W2, m1_b2,
             bn1_g, bn1_b, final_relu=True)

    p = _sc_agg(h, src, dst, ep2)[:, :N]
    h = _mlp(h, p[0], p[1], m2_W1, m2_b1, m2_g, m2_bb, m2_W2, m2_b2,
             bn2_g, bn2_b, final_relu=True)

    p = _sc_agg(h, src, dst, ep3)[:, :N]
    h = _mlp(h, p[0], p[1], m3_W1, m3_b1, m3_g, m3_bb, m3_W2, m3_b2,
             bn3_g, bn3_b, final_relu=False)

    return _pool(h, batch)
